# baseline (speedup 1.0000x reference)
import functools

import jax
import jax.numpy as jnp
from jax import lax
from jax.experimental import pallas as pl
from jax.experimental.pallas import tpu as pltpu
from jax.experimental.pallas import tpu_sc as plsc

NC = 2
NS = 16
NW = NC * NS
CE = 128
CD = 64
CS = 128
EW_W = 16
HEADS = 8
DHEAD = 16
DMODEL = 128
ALPHA_MIX = 0.1
N_STEPS = 5


def _sc_mesh():
    return plsc.VectorSubcoreMesh(
        core_axis_name="c", subcore_axis_name="s", num_cores=NC,
        num_subcores=NS)


def _edge_prod(kmat, qmat, src3, dst3):
    cpt = src3.shape[1]
    ep = NW * cpt * CE

    @functools.partial(
        pl.kernel,
        out_type=jax.ShapeDtypeStruct((ep, DMODEL), jnp.float32),
        mesh=_sc_mesh(),
        scratch_types=[
            pltpu.VMEM((CE,), jnp.int32), pltpu.VMEM((CE,), jnp.int32),
            pltpu.VMEM((CE,), jnp.int32), pltpu.VMEM((CE,), jnp.int32),
            pltpu.VMEM((CE, DMODEL), jnp.float32),
            pltpu.VMEM((CE, DMODEL), jnp.float32),
            pltpu.VMEM((CE, DMODEL), jnp.float32),
            pltpu.VMEM((CE, DMODEL), jnp.float32),
            pltpu.SemaphoreType.DMA, pltpu.SemaphoreType.DMA,
            pltpu.SemaphoreType.DMA, pltpu.SemaphoreType.DMA,
        ],
    )
    def body(k_hbm, q_hbm, src_hbm, dst_hbm, out_hbm, si0, si1, di0, di1,
             kr0, kr1, qr0, qr1, ks0, ks1, qs0, qs1):
        cid = lax.axis_index("c")
        sid = lax.axis_index("s")
        wid = sid * NC + cid
        sidx = [si0, si1]
        didx = [di0, di1]
        krows = [kr0, kr1]
        qrows = [qr0, qr1]
        ksem = [ks0, ks1]
        qsem = [qs0, qs1]

        def start(b, j):
            pltpu.sync_copy(src_hbm.at[wid, j], sidx[b])
            pltpu.sync_copy(dst_hbm.at[wid, j], didx[b])
            pltpu.async_copy(k_hbm.at[sidx[b]], krows[b], ksem[b])
            pltpu.async_copy(q_hbm.at[didx[b]], qrows[b], qsem[b])

        start(0, 0)
        start(1, 1)

        def outer(t, carry):
            j0 = t * 2
            for b in range(2):
                j = j0 + b
                pltpu.make_async_copy(
                    k_hbm.at[sidx[b]], krows[b], ksem[b]).wait()
                pltpu.make_async_copy(
                    q_hbm.at[didx[b]], qrows[b], qsem[b]).wait()

                def row(r, c2, _b=b):
                    for h in range(HEADS):
                        sl = pl.ds(h * DHEAD, DHEAD)
                        krows[_b][r, sl] = krows[_b][r, sl] * qrows[_b][r, sl]
                    return c2

                lax.fori_loop(0, CE, row, 0)
                base = pl.multiple_of((wid * cpt + j) * CE, 8)
                pltpu.sync_copy(krows[b], out_hbm.at[pl.ds(base, CE)])

                @pl.when(j + 2 < cpt)
                def _(_b=b, _j=j):
                    start(_b, _j + 2)
            return carry

        lax.fori_loop(0, cpt // 2, outer, 0)

    return body(kmat, qmat, src3, dst3)


def _scatter_es(es16, dst3, zeros16):
    npad = zeros16.shape[0]
    cpt = dst3.shape[1]
    rows_per = npad // NS

    @functools.partial(
        pl.kernel,
        out_type=jax.ShapeDtypeStruct((NC, npad, EW_W), jnp.float32),
        mesh=_sc_mesh(),
        scratch_types=[
            pltpu.VMEM((CS,), jnp.int32),
            pltpu.VMEM((CS, EW_W), jnp.float32),
            pltpu.VMEM_SHARED((npad, EW_W), jnp.float32),
        ],
    )
    def body(es_hbm, dst_hbm, z_hbm, out_hbm, didx, ebuf, acc):
        cid = lax.axis_index("c")
        sid = lax.axis_index("s")
        wid = sid * NC + cid
        off = sid * rows_per
        pltpu.sync_copy(z_hbm.at[pl.ds(off, rows_per)],
                        acc.at[pl.ds(off, rows_per)])
        plsc.subcore_barrier()

        def chunk(j, carry):
            pltpu.sync_copy(dst_hbm.at[wid, j], didx)
            base = pl.multiple_of((wid * cpt + j) * CS, 8)
            pltpu.sync_copy(es_hbm.at[pl.ds(base, CS)], ebuf)
            pltpu.sync_copy(ebuf, acc.at[didx], add=True)
            return carry

        lax.fori_loop(0, cpt, chunk, 0)
        plsc.subcore_barrier()
        pltpu.sync_copy(acc.at[pl.ds(off, rows_per)],
                        out_hbm.at[cid, pl.ds(off, rows_per)])

    return body(es16, dst3, zeros16)


def _diffuse(hmat, es16, src3, dst3, zeros_np, zeros16):
    npad = zeros_np.shape[0]
    cpt = src3.shape[1]
    rows_per = npad // NS
    fuse = zeros16 is not None

    out_type = [jax.ShapeDtypeStruct((NC, npad, DMODEL), jnp.float32)]
    scratch = (
        [pltpu.VMEM((CD,), jnp.int32)] * 4
        + [pltpu.VMEM((CD, DMODEL), jnp.float32)] * 2
        + [pltpu.VMEM((CD, EW_W), jnp.float32)] * 2
        + [pltpu.VMEM_SHARED((npad, DMODEL), jnp.float32)]
        + [pltpu.SemaphoreType.DMA] * 4
    )
    if fuse:
        out_type.append(jax.ShapeDtypeStruct((NC, npad, EW_W), jnp.float32))
        scratch.append(pltpu.VMEM_SHARED((npad, EW_W), jnp.float32))

    @functools.partial(
        pl.kernel,
        out_type=tuple(out_type),
        mesh=_sc_mesh(),
        scratch_types=tuple(scratch),
    )
    def body(*refs):
        if fuse:
            (h_hbm, es_hbm, src_hbm, dst_hbm, z_hbm, z16_hbm,
             out_hbm, w_hbm, si0, si1, di0, di1, hr0, hr1, eb0, eb1,
             acc, gs0, gs1, es0, es1, acc16) = refs
        else:
            (h_hbm, es_hbm, src_hbm, dst_hbm, z_hbm,
             out_hbm, si0, si1, di0, di1, hr0, hr1, eb0, eb1,
             acc, gs0, gs1, es0, es1) = refs
        cid = lax.axis_index("c")
        sid = lax.axis_index("s")
        wid = sid * NC + cid
        off = sid * rows_per
        sidx = [si0, si1]
        didx = [di0, di1]
        hrows = [hr0, hr1]
        ebuf = [eb0, eb1]
        gsem = [gs0, gs1]
        esem = [es0, es1]

        def start(b, j):
            pltpu.sync_copy(src_hbm.at[wid, j], sidx[b])
            pltpu.sync_copy(dst_hbm.at[wid, j], didx[b])
            pltpu.async_copy(h_hbm.at[sidx[b]], hrows[b], gsem[b])
            base = pl.multiple_of((wid * cpt + j) * CD, 8)
            pltpu.async_copy(es_hbm.at[pl.ds(base, CD)], ebuf[b], esem[b])

        start(0, 0)
        start(1, 1)
        pltpu.sync_copy(z_hbm.at[pl.ds(off, rows_per)],
                        acc.at[pl.ds(off, rows_per)])
        if fuse:
            pltpu.sync_copy(z16_hbm.at[pl.ds(off, rows_per)],
                            acc16.at[pl.ds(off, rows_per)])
        plsc.subcore_barrier()

        def outer(t, carry):
            j0 = t * 2
            for b in range(2):
                j = j0 + b
                pltpu.make_async_copy(
                    h_hbm.at[sidx[b]], hrows[b], gsem[b]).wait()
                pltpu.make_async_copy(
                    es_hbm.at[pl.ds(0, CD)], ebuf[b], esem[b]).wait()

                def row(r, c2, _b=b):
                    evec = ebuf[_b][r, :]
                    for h in range(HEADS):
                        sl = pl.ds(h * DHEAD, DHEAD)
                        w = jnp.broadcast_to(evec[h], (DHEAD,))
                        hrows[_b][r, sl] = hrows[_b][r, sl] * w
                    return c2

                lax.fori_loop(0, CD, row, 0)
                if fuse:
                    pltpu.sync_copy(ebuf[b], acc16.at[didx[b]], add=True)
                pltpu.sync_copy(hrows[b], acc.at[didx[b]], add=True)

                @pl.when(j + 2 < cpt)
                def _(_b=b, _j=j):
                    start(_b, _j + 2)
            return carry

        lax.fori_loop(0, cpt // 2, outer, 0)
        plsc.subcore_barrier()
        pltpu.sync_copy(acc.at[pl.ds(off, rows_per)],
                        out_hbm.at[cid, pl.ds(off, rows_per)])
        if fuse:
            pltpu.sync_copy(acc16.at[pl.ds(off, rows_per)],
                            w_hbm.at[cid, pl.ds(off, rows_per)])

    if fuse:
        return body(hmat, es16, src3, dst3, zeros_np, zeros16)
    return body(hmat, es16, src3, dst3, zeros_np)


def _project(hs_pad, wqt, bq, wkt, bk, wvt, bv):
    npad = hs_pad.shape[0]
    blk = 512

    def body(x_ref, wq_ref, bq_ref, wk_ref, bk_ref, wv_ref, bv_ref,
             q_ref, k_ref, v_ref):
        x = x_ref[...]
        q = jnp.dot(x, wq_ref[...], preferred_element_type=jnp.float32)
        q_ref[...] = (q + bq_ref[...]) * (1.0 / (DHEAD ** 0.5))
        k_ref[...] = jnp.dot(
            x, wk_ref[...], preferred_element_type=jnp.float32) + bk_ref[...]
        v_ref[...] = jnp.dot(
            x, wv_ref[...], preferred_element_type=jnp.float32) + bv_ref[...]

    full = pl.BlockSpec((DMODEL, DMODEL), lambda i: (0, 0))
    bias = pl.BlockSpec((1, DMODEL), lambda i: (0, 0))
    rows = pl.BlockSpec((blk, DMODEL), lambda i: (i, 0))
    out_sd = jax.ShapeDtypeStruct((npad, DMODEL), jnp.float32)
    return pl.pallas_call(
        body,
        grid=(npad // blk,),
        in_specs=[rows, full, bias, full, bias, full, bias],
        out_specs=[rows, rows, rows],
        out_shape=[out_sd, out_sd, out_sd],
    )(hs_pad, wqt, bq.reshape(1, DMODEL), wkt, bk.reshape(1, DMODEL),
      wvt, bv.reshape(1, DMODEL))


def _score(prod, e_real):
    ep = prod.shape[0]
    blk = 2048
    bsel = jnp.repeat(jnp.eye(HEADS, dtype=jnp.float32), DHEAD, axis=0)

    def body(p_ref, bs_ref, es_ref):
        i = pl.program_id(0)
        s = jnp.dot(p_ref[...], bs_ref[...],
                    preferred_element_type=jnp.float32)
        es = jnp.exp(s)
        rowid = i * blk + lax.broadcasted_iota(jnp.int32, (blk, 1), 0)
        es = jnp.where(rowid < e_real, es, 0.0)
        es_ref[...] = jnp.concatenate([es, es], axis=1)

    return pl.pallas_call(
        body,
        grid=(ep // blk,),
        in_specs=[
            pl.BlockSpec((blk, DMODEL), lambda i: (i, 0)),
            pl.BlockSpec((DMODEL, HEADS), lambda i: (0, 0)),
        ],
        out_specs=pl.BlockSpec((blk, EW_W), lambda i: (i, 0)),
        out_shape=jax.ShapeDtypeStruct((ep, EW_W), jnp.float32),
    )(prod, bsel)


def _lerp(s0, s1, w0, w1, vmat):
    npad = vmat.shape[0]
    blk = 512
    bselt = jnp.repeat(jnp.eye(HEADS, dtype=jnp.float32), DHEAD, axis=1)

    def body(s0_ref, s1_ref, w0_ref, w1_ref, bst_ref, v_ref, out_ref):
        num = s0_ref[...] + s1_ref[...]
        w8 = (w0_ref[...] + w1_ref[...])[:, :HEADS]
        den = jnp.dot(w8, bst_ref[...],
                      preferred_element_type=jnp.float32) + 1e-9
        out_ref[...] = ((1.0 - ALPHA_MIX) * num / den
                        + ALPHA_MIX * v_ref[...])

    rows = pl.BlockSpec((blk, DMODEL), lambda i: (i, 0))
    rows_w = pl.BlockSpec((blk, EW_W), lambda i: (i, 0))
    bst = pl.BlockSpec((HEADS, DMODEL), lambda i: (0, 0))
    return pl.pallas_call(
        body,
        grid=(npad // blk,),
        in_specs=[rows, rows, rows_w, rows_w, bst, rows],
        out_specs=rows,
        out_shape=jax.ShapeDtypeStruct((npad, DMODEL), jnp.float32),
    )(s0, s1, w0, w1, bselt, vmat)


def kernel(hidden_states, attention_mask, Wq, bq, Wk, bk, Wv, bv, edge_index):
    del attention_mask
    b, s, d = hidden_states.shape
    n = b * s
    npad = ((n + 2560 - 1) // 2560) * 2560
    e = edge_index.shape[1]

    cpt_e = -(-e // (NW * CE))
    cpt_e += cpt_e % 2
    ep_e = NW * cpt_e * CE
    cpt_d = -(-e // (NW * CD))
    cpt_d += cpt_d % 2
    ep_d = NW * cpt_d * CD
    cpt_s = -(-e // (NW * CS))
    ep_s = NW * cpt_s * CS
    ep = max(ep_e, ep_d, ep_s)

    hs2 = hidden_states.reshape(n, d)
    hs_pad = jnp.pad(hs2, ((0, npad - n), (0, 0)))
    q, k, v = _project(hs_pad, Wq.T, bq, Wk.T, bk, Wv.T, bv)

    src_flat = jnp.pad(edge_index[0], (0, ep - e))
    dst_flat = jnp.pad(edge_index[1], (0, ep - e))
    src3e = src_flat[:ep_e].reshape(NW, cpt_e, CE)
    dst3e = dst_flat[:ep_e].reshape(NW, cpt_e, CE)
    src3d = src_flat[:ep_d].reshape(NW, cpt_d, CD)
    dst3d = dst_flat[:ep_d].reshape(NW, cpt_d, CD)
    dst3s = dst_flat[:ep_s].reshape(NW, cpt_s, CS)

    prod = _edge_prod(k, q, src3e, dst3e)
    es16_full = _score(prod, e)

    def crop(a, m):
        return a[:m] if m < a.shape[0] else jnp.pad(
            a, ((0, m - a.shape[0]), (0, 0)))

    es16 = crop(es16_full, ep_d)
    es16_s = crop(es16_full, ep_s)

    zeros_np = jnp.zeros((npad, DMODEL), jnp.float32)
    zeros16 = jnp.zeros((npad, EW_W), jnp.float32)

    wpart = _scatter_es(es16_s, dst3s, zeros16)
    w0, w1 = wpart[0], wpart[1]

    h = v
    for step in range(N_STEPS):
        (spart,) = _diffuse(h, es16, src3d, dst3d, zeros_np, None)
        h = _lerp(spart[0], spart[1], w0, w1, v)

    return h[:n].reshape(b, s, d)


# speedup vs baseline: 1.0649x; 1.0649x over previous
import functools

import jax
import jax.numpy as jnp
from jax import lax
from jax.experimental import pallas as pl
from jax.experimental.pallas import tpu as pltpu
from jax.experimental.pallas import tpu_sc as plsc

NC = 2
NS = 16
NW = NC * NS
CE = 128
CD = 64
EW_W = 16
HEADS = 8
DHEAD = 16
DMODEL = 128
ALPHA_MIX = 0.1
N_STEPS = 5


def _sc_mesh():
    return plsc.VectorSubcoreMesh(
        core_axis_name="c", subcore_axis_name="s", num_cores=NC,
        num_subcores=NS)


def _edge_prod(kmat, qmat, src3, dst3):
    cpt = src3.shape[1]
    ep = NW * cpt * CE

    @functools.partial(
        pl.kernel,
        out_type=jax.ShapeDtypeStruct((ep, DMODEL), jnp.float32),
        mesh=_sc_mesh(),
        scratch_types=[
            pltpu.VMEM((CE,), jnp.int32), pltpu.VMEM((CE,), jnp.int32),
            pltpu.VMEM((CE,), jnp.int32), pltpu.VMEM((CE,), jnp.int32),
            pltpu.VMEM((CE, DMODEL), jnp.float32),
            pltpu.VMEM((CE, DMODEL), jnp.float32),
            pltpu.VMEM((CE, DMODEL), jnp.float32),
            pltpu.VMEM((CE, DMODEL), jnp.float32),
            pltpu.SemaphoreType.DMA, pltpu.SemaphoreType.DMA,
            pltpu.SemaphoreType.DMA, pltpu.SemaphoreType.DMA,
        ],
    )
    def body(k_hbm, q_hbm, src_hbm, dst_hbm, out_hbm, si0, si1, di0, di1,
             kr0, kr1, qr0, qr1, ks0, ks1, qs0, qs1):
        cid = lax.axis_index("c")
        sid = lax.axis_index("s")
        wid = sid * NC + cid
        sidx = [si0, si1]
        didx = [di0, di1]
        krows = [kr0, kr1]
        qrows = [qr0, qr1]
        ksem = [ks0, ks1]
        qsem = [qs0, qs1]

        def start(b, j):
            pltpu.sync_copy(src_hbm.at[wid, j], sidx[b])
            pltpu.sync_copy(dst_hbm.at[wid, j], didx[b])
            pltpu.async_copy(k_hbm.at[sidx[b]], krows[b], ksem[b])
            pltpu.async_copy(q_hbm.at[didx[b]], qrows[b], qsem[b])

        start(0, 0)
        start(1, 1)

        def outer(t, carry):
            j0 = t * 2
            for b in range(2):
                j = j0 + b
                pltpu.make_async_copy(
                    k_hbm.at[sidx[b]], krows[b], ksem[b]).wait()
                pltpu.make_async_copy(
                    q_hbm.at[didx[b]], qrows[b], qsem[b]).wait()

                def row(r, c2, _b=b):
                    for h in range(HEADS):
                        sl = pl.ds(h * DHEAD, DHEAD)
                        krows[_b][r, sl] = krows[_b][r, sl] * qrows[_b][r, sl]
                    return c2

                lax.fori_loop(0, CE, row, 0)
                base = pl.multiple_of((wid * cpt + j) * CE, 8)
                pltpu.sync_copy(krows[b], out_hbm.at[pl.ds(base, CE)])

                @pl.when(j + 2 < cpt)
                def _(_b=b, _j=j):
                    start(_b, _j + 2)
            return carry

        lax.fori_loop(0, cpt // 2, outer, 0)

    return body(kmat, qmat, src3, dst3)


def _scatter_es(es16, dst3, zeros16):
    npad = zeros16.shape[0]
    cpt = dst3.shape[1]
    rows_per = npad // NS

    @functools.partial(
        pl.kernel,
        out_type=jax.ShapeDtypeStruct((NC, npad, EW_W), jnp.float32),
        mesh=_sc_mesh(),
        scratch_types=[
            pltpu.VMEM((CD,), jnp.int32),
            pltpu.VMEM((CD, EW_W), jnp.float32),
            pltpu.VMEM_SHARED((npad, EW_W), jnp.float32),
        ],
    )
    def body(es_hbm, dst_hbm, z_hbm, out_hbm, didx, ebuf, acc):
        cid = lax.axis_index("c")
        sid = lax.axis_index("s")
        wid = sid * NC + cid
        off = sid * rows_per
        pltpu.sync_copy(z_hbm.at[pl.ds(off, rows_per)],
                        acc.at[pl.ds(off, rows_per)])
        plsc.subcore_barrier()

        def chunk(j, carry):
            pltpu.sync_copy(dst_hbm.at[wid, j], didx)
            base = pl.multiple_of((wid * cpt + j) * CD, 8)
            pltpu.sync_copy(es_hbm.at[pl.ds(base, CD)], ebuf)
            pltpu.sync_copy(ebuf, acc.at[didx], add=True)
            return carry

        lax.fori_loop(0, cpt, chunk, 0)
        plsc.subcore_barrier()
        pltpu.sync_copy(acc.at[pl.ds(off, rows_per)],
                        out_hbm.at[cid, pl.ds(off, rows_per)])

    return body(es16, dst3, zeros16)


def _diffuse(hmat, es16, src3, dst3, zeros_np, zeros16):
    npad = zeros_np.shape[0]
    cpt = src3.shape[1]
    rows_per = npad // NS
    fuse = zeros16 is not None

    out_type = [jax.ShapeDtypeStruct((NC, npad, DMODEL), jnp.float32)]
    scratch = (
        [pltpu.VMEM((CD,), jnp.int32)] * 4
        + [pltpu.VMEM((CD, DMODEL), jnp.float32)] * 2
        + [pltpu.VMEM((CD, EW_W), jnp.float32)] * 2
        + [pltpu.VMEM_SHARED((npad, DMODEL), jnp.float32)]
        + [pltpu.SemaphoreType.DMA] * 4
    )
    if fuse:
        out_type.append(jax.ShapeDtypeStruct((NC, npad, EW_W), jnp.float32))
        scratch.append(pltpu.VMEM_SHARED((npad, EW_W), jnp.float32))

    @functools.partial(
        pl.kernel,
        out_type=tuple(out_type),
        mesh=_sc_mesh(),
        scratch_types=tuple(scratch),
    )
    def body(*refs):
        if fuse:
            (h_hbm, es_hbm, src_hbm, dst_hbm, z_hbm, z16_hbm,
             out_hbm, w_hbm, si0, si1, di0, di1, hr0, hr1, eb0, eb1,
             acc, gs0, gs1, es0, es1, acc16) = refs
        else:
            (h_hbm, es_hbm, src_hbm, dst_hbm, z_hbm,
             out_hbm, si0, si1, di0, di1, hr0, hr1, eb0, eb1,
             acc, gs0, gs1, es0, es1) = refs
        cid = lax.axis_index("c")
        sid = lax.axis_index("s")
        wid = sid * NC + cid
        off = sid * rows_per
        sidx = [si0, si1]
        didx = [di0, di1]
        hrows = [hr0, hr1]
        ebuf = [eb0, eb1]
        gsem = [gs0, gs1]
        esem = [es0, es1]

        def start(b, j):
            pltpu.sync_copy(src_hbm.at[wid, j], sidx[b])
            pltpu.sync_copy(dst_hbm.at[wid, j], didx[b])
            pltpu.async_copy(h_hbm.at[sidx[b]], hrows[b], gsem[b])
            base = pl.multiple_of((wid * cpt + j) * CD, 8)
            pltpu.async_copy(es_hbm.at[pl.ds(base, CD)], ebuf[b], esem[b])

        start(0, 0)
        start(1, 1)
        pltpu.sync_copy(z_hbm.at[pl.ds(off, rows_per)],
                        acc.at[pl.ds(off, rows_per)])
        if fuse:
            pltpu.sync_copy(z16_hbm.at[pl.ds(off, rows_per)],
                            acc16.at[pl.ds(off, rows_per)])
        plsc.subcore_barrier()

        def outer(t, carry):
            j0 = t * 2
            for b in range(2):
                j = j0 + b
                pltpu.make_async_copy(
                    h_hbm.at[sidx[b]], hrows[b], gsem[b]).wait()
                pltpu.make_async_copy(
                    es_hbm.at[pl.ds(0, CD)], ebuf[b], esem[b]).wait()

                def row(r, c2, _b=b):
                    evec = ebuf[_b][r, :]
                    for h in range(HEADS):
                        sl = pl.ds(h * DHEAD, DHEAD)
                        w = jnp.broadcast_to(evec[h], (DHEAD,))
                        hrows[_b][r, sl] = hrows[_b][r, sl] * w
                    return c2

                lax.fori_loop(0, CD, row, 0)
                if fuse:
                    pltpu.sync_copy(ebuf[b], acc16.at[didx[b]], add=True)
                pltpu.sync_copy(hrows[b], acc.at[didx[b]], add=True)

                @pl.when(j + 2 < cpt)
                def _(_b=b, _j=j):
                    start(_b, _j + 2)
            return carry

        lax.fori_loop(0, cpt // 2, outer, 0)
        plsc.subcore_barrier()
        pltpu.sync_copy(acc.at[pl.ds(off, rows_per)],
                        out_hbm.at[cid, pl.ds(off, rows_per)])
        if fuse:
            pltpu.sync_copy(acc16.at[pl.ds(off, rows_per)],
                            w_hbm.at[cid, pl.ds(off, rows_per)])

    if fuse:
        return body(hmat, es16, src3, dst3, zeros_np, zeros16)
    return body(hmat, es16, src3, dst3, zeros_np)


def _project(hs_pad, wqt, bq, wkt, bk, wvt, bv):
    npad = hs_pad.shape[0]
    blk = 512

    def body(x_ref, wq_ref, bq_ref, wk_ref, bk_ref, wv_ref, bv_ref,
             q_ref, k_ref, v_ref):
        x = x_ref[...]
        q = jnp.dot(x, wq_ref[...], preferred_element_type=jnp.float32)
        q_ref[...] = (q + bq_ref[...]) * (1.0 / (DHEAD ** 0.5))
        k_ref[...] = jnp.dot(
            x, wk_ref[...], preferred_element_type=jnp.float32) + bk_ref[...]
        v_ref[...] = jnp.dot(
            x, wv_ref[...], preferred_element_type=jnp.float32) + bv_ref[...]

    full = pl.BlockSpec((DMODEL, DMODEL), lambda i: (0, 0))
    bias = pl.BlockSpec((1, DMODEL), lambda i: (0, 0))
    rows = pl.BlockSpec((blk, DMODEL), lambda i: (i, 0))
    out_sd = jax.ShapeDtypeStruct((npad, DMODEL), jnp.float32)
    return pl.pallas_call(
        body,
        grid=(npad // blk,),
        in_specs=[rows, full, bias, full, bias, full, bias],
        out_specs=[rows, rows, rows],
        out_shape=[out_sd, out_sd, out_sd],
    )(hs_pad, wqt, bq.reshape(1, DMODEL), wkt, bk.reshape(1, DMODEL),
      wvt, bv.reshape(1, DMODEL))


def _score(prod, e_real):
    ep = prod.shape[0]
    blk = 2048
    bsel = jnp.repeat(jnp.eye(HEADS, dtype=jnp.float32), DHEAD, axis=0)

    def body(p_ref, bs_ref, es_ref):
        i = pl.program_id(0)
        s = jnp.dot(p_ref[...], bs_ref[...],
                    preferred_element_type=jnp.float32)
        es = jnp.exp(s)
        rowid = i * blk + lax.broadcasted_iota(jnp.int32, (blk, 1), 0)
        es = jnp.where(rowid < e_real, es, 0.0)
        es_ref[...] = jnp.concatenate([es, es], axis=1)

    return pl.pallas_call(
        body,
        grid=(ep // blk,),
        in_specs=[
            pl.BlockSpec((blk, DMODEL), lambda i: (i, 0)),
            pl.BlockSpec((DMODEL, HEADS), lambda i: (0, 0)),
        ],
        out_specs=pl.BlockSpec((blk, EW_W), lambda i: (i, 0)),
        out_shape=jax.ShapeDtypeStruct((ep, EW_W), jnp.float32),
    )(prod, bsel)


def _lerp(s0, s1, w0, w1, vmat):
    npad = vmat.shape[0]
    blk = 512
    bselt = jnp.repeat(jnp.eye(HEADS, dtype=jnp.float32), DHEAD, axis=1)

    def body(s0_ref, s1_ref, w0_ref, w1_ref, bst_ref, v_ref, out_ref):
        num = s0_ref[...] + s1_ref[...]
        w8 = (w0_ref[...] + w1_ref[...])[:, :HEADS]
        den = jnp.dot(w8, bst_ref[...],
                      preferred_element_type=jnp.float32) + 1e-9
        out_ref[...] = ((1.0 - ALPHA_MIX) * num / den
                        + ALPHA_MIX * v_ref[...])

    rows = pl.BlockSpec((blk, DMODEL), lambda i: (i, 0))
    rows_w = pl.BlockSpec((blk, EW_W), lambda i: (i, 0))
    bst = pl.BlockSpec((HEADS, DMODEL), lambda i: (0, 0))
    return pl.pallas_call(
        body,
        grid=(npad // blk,),
        in_specs=[rows, rows, rows_w, rows_w, bst, rows],
        out_specs=rows,
        out_shape=jax.ShapeDtypeStruct((npad, DMODEL), jnp.float32),
    )(s0, s1, w0, w1, bselt, vmat)


def kernel(hidden_states, attention_mask, Wq, bq, Wk, bk, Wv, bv, edge_index):
    del attention_mask
    b, s, d = hidden_states.shape
    n = b * s
    npad = ((n + 2560 - 1) // 2560) * 2560
    e = edge_index.shape[1]

    cpt_e = -(-e // (NW * CE))
    cpt_e += cpt_e % 2
    ep_e = NW * cpt_e * CE
    cpt_d = -(-e // (NW * CD))
    cpt_d += cpt_d % 2
    ep_d = NW * cpt_d * CD
    ep = max(ep_e, ep_d)

    hs2 = hidden_states.reshape(n, d)
    hs_pad = jnp.pad(hs2, ((0, npad - n), (0, 0)))
    q, k, v = _project(hs_pad, Wq.T, bq, Wk.T, bk, Wv.T, bv)

    src_flat = jnp.pad(edge_index[0], (0, ep - e))
    dst_flat = jnp.pad(edge_index[1], (0, ep - e))
    src3e = src_flat[:ep_e].reshape(NW, cpt_e, CE)
    dst3e = dst_flat[:ep_e].reshape(NW, cpt_e, CE)
    src3d = src_flat[:ep_d].reshape(NW, cpt_d, CD)
    dst3d = dst_flat[:ep_d].reshape(NW, cpt_d, CD)

    prod = _edge_prod(k, q, src3e, dst3e)
    es16 = _score(prod, e)
    es16 = es16[:ep_d] if ep_d < es16.shape[0] else jnp.pad(
        es16, ((0, ep_d - es16.shape[0]), (0, 0)))

    zeros_np = jnp.zeros((npad, DMODEL), jnp.float32)
    zeros16 = jnp.zeros((npad, EW_W), jnp.float32)

    wpart = _scatter_es(es16, dst3d, zeros16)
    w0, w1 = wpart[0], wpart[1]

    h = v
    for step in range(N_STEPS):
        (spart,) = _diffuse(h, es16, src3d, dst3d, zeros_np, None)
        h = _lerp(spart[0], spart[1], w0, w1, v)

    return h[:n].reshape(b, s, d)
